# baseline (speedup 1.0000x reference)
import jax
import jax.numpy as jnp
from jax import lax
from jax.experimental import pallas as pl
from jax.experimental.pallas import tpu as pltpu

IMAGE_SIZE = 1024.0
K_KEEP = 6000
NUM_OUT = 1000
IOU_T = 0.7

N_IN = 20000
N_PAD = 20480
ROWS = N_PAD // 128
CROWS = 48
CPLANE = CROWS + 1
NEG = -1e10
HIGHEST = lax.Precision.HIGHEST


def _decode(a, d):
    h = a[2] - a[0]
    w = a[3] - a[1]
    cy = a[0] + 0.5 * h
    cx = a[1] + 0.5 * w
    cy = cy + (d[0] * 0.1) * h
    cx = cx + (d[1] * 0.1) * w
    h = h * jnp.exp(d[2] * 0.2)
    w = w * jnp.exp(d[3] * 0.2)
    y1 = cy - 0.5 * h
    x1 = cx - 0.5 * w
    y2 = y1 + h
    x2 = x1 + w
    Y1 = jnp.clip(y1, 0.0, IMAGE_SIZE) / IMAGE_SIZE
    X1 = jnp.clip(x1, 0.0, IMAGE_SIZE) / IMAGE_SIZE
    Y2 = jnp.clip(y2, 0.0, IMAGE_SIZE) / IMAGE_SIZE
    X2 = jnp.clip(x2, 0.0, IMAGE_SIZE) / IMAGE_SIZE
    return Y1, X1, Y2, X2


def _topk_mask(s, fi):
    u = lax.bitcast_convert_type(s, jnp.int32)
    active = u >= 0
    above = jnp.zeros_like(active)
    krem = jnp.float32(K_KEEP)
    for bit in range(30, -1, -1):
        bitset = ((u >> bit) & 1) == 1
        hi = active & bitset
        c = jnp.sum(hi.astype(jnp.float32))
        pred = krem <= c
        npred = jnp.logical_not(pred)
        above = above | (hi & npred)
        krem = jnp.where(pred, krem, krem - c)
        active = (hi & pred) | (active & jnp.logical_not(bitset) & npred)
    sel = above
    k2 = krem
    for bit in range(14, -1, -1):
        bit0 = ((fi >> bit) & 1) == 0
        lo = active & bit0
        c = jnp.sum(lo.astype(jnp.float32))
        pred = k2 <= c
        npred = jnp.logical_not(pred)
        sel = sel | (lo & npred)
        k2 = jnp.where(pred, k2, k2 - c)
        active = (lo & pred) | (active & jnp.logical_not(bit0) & npred)
    return sel | active


def _positions(sel):
    self = sel.astype(jnp.float32)
    tri128 = (lax.broadcasted_iota(jnp.int32, (128, 128), 0)
              < lax.broadcasted_iota(jnp.int32, (128, 128), 1)
              ).astype(jnp.float32)
    within = lax.dot(self, tri128, precision=HIGHEST)
    rowtot = lax.dot(self, jnp.ones((128, 1), jnp.float32),
                     precision=HIGHEST)
    triR = (lax.broadcasted_iota(jnp.int32, (ROWS, ROWS), 1)
            < lax.broadcasted_iota(jnp.int32, (ROWS, ROWS), 0)
            ).astype(jnp.float32)
    rowbase = lax.dot(triR, rowtot, precision=HIGHEST)
    p = (rowbase + within).astype(jnp.int32)
    return jnp.where(sel, p, jnp.int32(1 << 20))


def _nms_step(sw, Y1, X1, Y2, X2, areas, fi, big):
    m = jnp.max(sw, keepdims=True)
    alive = m > -1e9
    j = jnp.min(jnp.where(sw == m, fi, big), keepdims=True)
    emask = fi == j
    ef = emask.astype(jnp.float32)
    b0 = jnp.sum(ef * Y1, keepdims=True)
    b1 = jnp.sum(ef * X1, keepdims=True)
    b2 = jnp.sum(ef * Y2, keepdims=True)
    b3 = jnp.sum(ef * X2, keepdims=True)
    area_j = (b2 - b0) * (b3 - b1)
    yy1 = jnp.maximum(b0, Y1)
    xx1 = jnp.maximum(b1, X1)
    yy2 = jnp.minimum(b2, Y2)
    xx2 = jnp.minimum(b3, X2)
    inter = jnp.maximum(yy2 - yy1, 0.0) * jnp.maximum(xx2 - xx1, 0.0)
    iou = inter / (area_j + areas - inter + 1e-8)
    supp = (iou > IOU_T) | emask
    sw = jnp.where(alive & supp, NEG, sw)
    return sw, alive, (b0, b1, b2, b3)


def _proposal_kernel(scores_ref, deltas_ref, anchors_ref, out_ref,
                     planesA, planesB, pA, pB, compA, compB):
    a = anchors_ref[:]
    ri = lax.broadcasted_iota(jnp.int32, (ROWS, 128), 0)
    li = lax.broadcasted_iota(jnp.int32, (ROWS, 128), 1)
    fi = ri * 128 + li
    lane256 = lax.broadcasted_iota(jnp.int32, (128, 256), 1)
    big = jnp.int32(1 << 30)
    oiota = (lax.broadcasted_iota(jnp.int32, (8, 128), 0) * 128
             + lax.broadcasted_iota(jnp.int32, (8, 128), 1))
    zeros8 = jnp.zeros((8, 128), jnp.float32)

    for img, planes, pref, comp in ((0, planesA, pA, compA),
                                    (1, planesB, pB, compB)):
        s = scores_ref[img]
        d = deltas_ref[img]
        Y1, X1, Y2, X2 = _decode(a, d)
        sel = _topk_mask(s, fi)
        pref[...] = _positions(sel)
        planes[0:ROWS, :] = jnp.where(sel, s, 0.0)
        planes[ROWS:2 * ROWS, :] = Y1
        planes[2 * ROWS:3 * ROWS, :] = X1
        planes[3 * ROWS:4 * ROWS, :] = Y2
        planes[4 * ROWS:5 * ROWS, :] = X2
        comp[...] = jnp.zeros((5 * CPLANE, 128), jnp.float32)

    def crow(r, _):
        for planes, pref, comp in ((planesA, pA, compA), (planesB, pB, compB)):
            p_row = pref[pl.ds(r, 1), :]
            q = jnp.minimum(jnp.min(p_row) >> 7, jnp.int32(CROWS - 1))
            p_local = p_row - q * 128
            p_col = jnp.swapaxes(p_local, 0, 1)
            oh = (p_col == lane256).astype(jnp.float32)
            rows = [planes[pl.ds(c * ROWS + r, 1), :] for c in range(5)]
            data = jnp.concatenate(rows, axis=0)
            contrib = lax.dot(data, oh, precision=HIGHEST)
            for c in range(5):
                i0 = c * CPLANE
                comp[pl.ds(i0 + q, 1), :] += contrib[c:c + 1, 0:128]
                comp[pl.ds(i0 + q + 1, 1), :] += contrib[c:c + 1, 128:256]
        return 0

    lax.fori_loop(0, ROWS, crow, 0)

    fic = (lax.broadcasted_iota(jnp.int32, (CROWS, 128), 0) * 128
           + lax.broadcasted_iota(jnp.int32, (CROWS, 128), 1))
    st = []
    for comp in (compA, compB):
        pls = [comp[c * CPLANE:c * CPLANE + CROWS, :] for c in range(5)]
        swc, y1c, x1c, y2c, x2c = pls
        areas = (y2c - y1c) * (x2c - x1c)
        st.append((y1c, x1c, y2c, x2c, areas))
        del swc

    swA0 = compA[0:CROWS, :]
    swB0 = compB[0:CROWS, :]

    def body(i, carry):
        swA, swB, oA0, oA1, oA2, oA3, oB0, oB1, oB2, oB3 = carry
        swA, aliveA, bA = _nms_step(swA, *st[0], fic, big)
        swB, aliveB, bB = _nms_step(swB, *st[1], fic, big)
        omA = aliveA & (oiota == i)
        omB = aliveB & (oiota == i)
        oA0 = jnp.where(omA, bA[0], oA0)
        oA1 = jnp.where(omA, bA[1], oA1)
        oA2 = jnp.where(omA, bA[2], oA2)
        oA3 = jnp.where(omA, bA[3], oA3)
        oB0 = jnp.where(omB, bB[0], oB0)
        oB1 = jnp.where(omB, bB[1], oB1)
        oB2 = jnp.where(omB, bB[2], oB2)
        oB3 = jnp.where(omB, bB[3], oB3)
        return (swA, swB, oA0, oA1, oA2, oA3, oB0, oB1, oB2, oB3)

    init = (swA0, swB0) + (zeros8,) * 8
    res = lax.fori_loop(0, NUM_OUT, body, init)
    for b in range(2):
        for c in range(4):
            out_ref[b, c] = res[2 + 4 * b + c]


def kernel(rpn_class, rpn_bbox, anchors):
    B = rpn_class.shape[0]
    pad = N_PAD - N_IN
    scores = jnp.pad(rpn_class[:, :, 1], ((0, 0), (0, pad)),
                     constant_values=-1.0).reshape(B, ROWS, 128)
    deltas = jnp.pad(jnp.transpose(rpn_bbox, (0, 2, 1)),
                     ((0, 0), (0, 0), (0, pad))).reshape(B, 4, ROWS, 128)
    anc = jnp.pad(anchors.T, ((0, 0), (0, pad))).reshape(4, ROWS, 128)

    out = pl.pallas_call(
        _proposal_kernel,
        in_specs=[
            pl.BlockSpec((B, ROWS, 128), lambda: (0, 0, 0)),
            pl.BlockSpec((B, 4, ROWS, 128), lambda: (0, 0, 0, 0)),
            pl.BlockSpec((4, ROWS, 128), lambda: (0, 0, 0)),
        ],
        out_specs=pl.BlockSpec((B, 4, 8, 128), lambda: (0, 0, 0, 0)),
        out_shape=jax.ShapeDtypeStruct((B, 4, 8, 128), jnp.float32),
        scratch_shapes=[
            pltpu.VMEM((5 * ROWS, 128), jnp.float32),
            pltpu.VMEM((5 * ROWS, 128), jnp.float32),
            pltpu.VMEM((ROWS, 128), jnp.int32),
            pltpu.VMEM((ROWS, 128), jnp.int32),
            pltpu.VMEM((5 * CPLANE, 128), jnp.float32),
            pltpu.VMEM((5 * CPLANE, 128), jnp.float32),
        ],
    )(scores, deltas, anc)
    return out.reshape(B, 4, 1024)[:, :, :NUM_OUT].transpose(0, 2, 1)


# speedup vs baseline: 1.0477x; 1.0477x over previous
import jax
import jax.numpy as jnp
from jax import lax
from jax.experimental import pallas as pl
from jax.experimental.pallas import tpu as pltpu

IMAGE_SIZE = 1024.0
K_KEEP = 6000
NUM_OUT = 1000
IOU_T = 0.7

N_IN = 20000
N_PAD = 20480
ROWS = N_PAD // 128
CROWS = 48
CPLANE = CROWS + 1
NEG = -1e10
HIGHEST = lax.Precision.HIGHEST


def _decode(a, d):
    h = a[2] - a[0]
    w = a[3] - a[1]
    cy = a[0] + 0.5 * h
    cx = a[1] + 0.5 * w
    cy = cy + (d[0] * 0.1) * h
    cx = cx + (d[1] * 0.1) * w
    h = h * jnp.exp(d[2] * 0.2)
    w = w * jnp.exp(d[3] * 0.2)
    y1 = cy - 0.5 * h
    x1 = cx - 0.5 * w
    y2 = y1 + h
    x2 = x1 + w
    Y1 = jnp.clip(y1, 0.0, IMAGE_SIZE) / IMAGE_SIZE
    X1 = jnp.clip(x1, 0.0, IMAGE_SIZE) / IMAGE_SIZE
    Y2 = jnp.clip(y2, 0.0, IMAGE_SIZE) / IMAGE_SIZE
    X2 = jnp.clip(x2, 0.0, IMAGE_SIZE) / IMAGE_SIZE
    return Y1, X1, Y2, X2


def _topk_mask(s, fi):
    u = lax.bitcast_convert_type(s, jnp.int32)
    active = u >= 0
    above = jnp.zeros_like(active)
    krem = jnp.full((1, 1), K_KEEP, jnp.float32)
    for bit in range(30, -1, -1):
        bitset = ((u >> bit) & 1) == 1
        hi = active & bitset
        c = jnp.sum(hi.astype(jnp.float32), keepdims=True)
        pred = krem <= c
        npred = jnp.logical_not(pred)
        above = above | (hi & npred)
        krem = jnp.where(pred, krem, krem - c)
        active = (hi & pred) | (active & jnp.logical_not(bitset) & npred)
    sel = above
    k2 = krem
    for bit in range(14, -1, -1):
        bit0 = ((fi >> bit) & 1) == 0
        lo = active & bit0
        c = jnp.sum(lo.astype(jnp.float32), keepdims=True)
        pred = k2 <= c
        npred = jnp.logical_not(pred)
        sel = sel | (lo & npred)
        k2 = jnp.where(pred, k2, k2 - c)
        active = (lo & pred) | (active & jnp.logical_not(bit0) & npred)
    return sel | active


def _positions(sel):
    self = sel.astype(jnp.float32)
    tri128 = (lax.broadcasted_iota(jnp.int32, (128, 128), 0)
              < lax.broadcasted_iota(jnp.int32, (128, 128), 1)
              ).astype(jnp.float32)
    within = lax.dot(self, tri128, precision=HIGHEST)
    rowtot = lax.dot(self, jnp.ones((128, 1), jnp.float32),
                     precision=HIGHEST)
    triR = (lax.broadcasted_iota(jnp.int32, (ROWS, ROWS), 1)
            < lax.broadcasted_iota(jnp.int32, (ROWS, ROWS), 0)
            ).astype(jnp.float32)
    rowbase = lax.dot(triR, rowtot, precision=HIGHEST)
    p = (rowbase + within).astype(jnp.int32)
    return jnp.where(sel, p, jnp.int32(1 << 20))


def _argmax(sw, fi, big):
    m = jnp.max(sw, keepdims=True)
    j = jnp.min(jnp.where(sw == m, fi, big), keepdims=True)
    return m, fi == j


def _extract(e, Y1, X1, Y2, X2):
    ef = e.astype(jnp.float32)
    return (jnp.sum(ef * Y1, keepdims=True), jnp.sum(ef * X1, keepdims=True),
            jnp.sum(ef * Y2, keepdims=True), jnp.sum(ef * X2, keepdims=True))


def _iou_vec(b, Y1, X1, Y2, X2, areas):
    area_j = (b[2] - b[0]) * (b[3] - b[1])
    yy1 = jnp.maximum(b[0], Y1)
    xx1 = jnp.maximum(b[1], X1)
    yy2 = jnp.minimum(b[2], Y2)
    xx2 = jnp.minimum(b[3], X2)
    inter = jnp.maximum(yy2 - yy1, 0.0) * jnp.maximum(xx2 - xx1, 0.0)
    return inter / (area_j + areas - inter + 1e-8)


def _nms_pair(sw, Y1, X1, Y2, X2, areas, fi, big):
    m1, e1 = _argmax(sw, fi, big)
    alive1 = m1 > -1e9
    b1 = _extract(e1, Y1, X1, Y2, X2)
    swx = jnp.where(e1, NEG, sw)
    m2, e2 = _argmax(swx, fi, big)
    alive2 = m2 > -1e9
    b2 = _extract(e2, Y1, X1, Y2, X2)
    iou1 = _iou_vec(b1, Y1, X1, Y2, X2, areas)
    iou2 = _iou_vec(b2, Y1, X1, Y2, X2, areas)
    iou12 = _iou_vec(b1, b2[0], b2[1], b2[2], b2[3],
                     (b2[2] - b2[0]) * (b2[3] - b2[1]))
    take2 = alive2 & jnp.logical_not(iou12 > IOU_T)
    supp = (alive1 & ((iou1 > IOU_T) | e1)) | (take2 & ((iou2 > IOU_T) | e2))
    sw = jnp.where(supp, NEG, sw)
    return sw, alive1, take2, b1, b2


def _proposal_kernel(scores_ref, deltas_ref, anchors_ref, out_ref,
                     planesA, planesB, pA, pB, compA, compB):
    a = anchors_ref[:]
    ri = lax.broadcasted_iota(jnp.int32, (ROWS, 128), 0)
    li = lax.broadcasted_iota(jnp.int32, (ROWS, 128), 1)
    fi = ri * 128 + li
    lane256 = lax.broadcasted_iota(jnp.int32, (128, 256), 1)
    big = jnp.int32(1 << 30)
    oiota = (lax.broadcasted_iota(jnp.int32, (8, 128), 0) * 128
             + lax.broadcasted_iota(jnp.int32, (8, 128), 1))
    zeros8 = jnp.zeros((8, 128), jnp.float32)

    for img, planes, pref, comp in ((0, planesA, pA, compA),
                                    (1, planesB, pB, compB)):
        s = scores_ref[img]
        d = deltas_ref[img]
        Y1, X1, Y2, X2 = _decode(a, d)
        sel = _topk_mask(s, fi)
        pref[...] = _positions(sel)
        planes[0:ROWS, :] = jnp.where(sel, s, 0.0)
        planes[ROWS:2 * ROWS, :] = Y1
        planes[2 * ROWS:3 * ROWS, :] = X1
        planes[3 * ROWS:4 * ROWS, :] = Y2
        planes[4 * ROWS:5 * ROWS, :] = X2
        comp[...] = jnp.zeros((5 * CPLANE, 128), jnp.float32)

    def crow(r, _):
        for planes, pref, comp in ((planesA, pA, compA), (planesB, pB, compB)):
            p_row = pref[pl.ds(r, 1), :]
            q = jnp.minimum(jnp.min(p_row) >> 7, jnp.int32(CROWS - 1))
            p_local = p_row - q * 128
            p_col = jnp.swapaxes(p_local, 0, 1)
            oh = (p_col == lane256).astype(jnp.float32)
            rows = [planes[pl.ds(c * ROWS + r, 1), :] for c in range(5)]
            data = jnp.concatenate(rows, axis=0)
            contrib = lax.dot(data, oh, precision=HIGHEST)
            for c in range(5):
                i0 = c * CPLANE
                comp[pl.ds(i0 + q, 1), :] += contrib[c:c + 1, 0:128]
                comp[pl.ds(i0 + q + 1, 1), :] += contrib[c:c + 1, 128:256]
        return 0

    lax.fori_loop(0, ROWS, crow, 0)

    fic = (lax.broadcasted_iota(jnp.int32, (CROWS, 128), 0) * 128
           + lax.broadcasted_iota(jnp.int32, (CROWS, 128), 1))
    st = []
    for comp in (compA, compB):
        pls = [comp[c * CPLANE:c * CPLANE + CROWS, :] for c in range(5)]
        swc, y1c, x1c, y2c, x2c = pls
        areas = (y2c - y1c) * (x2c - x1c)
        st.append((y1c, x1c, y2c, x2c, areas))
        del swc

    swA0 = compA[0:CROWS, :]
    swB0 = compB[0:CROWS, :]
    one11 = jnp.zeros((1, 1), jnp.int32)

    def cond(carry):
        cntA, cntB, alA, alB = carry[2], carry[3], carry[4], carry[5]
        goA = jnp.where((cntA < NUM_OUT) & (alA > 0), 1, 0)
        goB = jnp.where((cntB < NUM_OUT) & (alB > 0), 1, 0)
        return (goA + goB)[0, 0] > 0

    def body(carry):
        (swA, swB, cntA, cntB, alA, alB,
         oA0, oA1, oA2, oA3, oB0, oB1, oB2, oB3) = carry
        swA, a1A, t2A, bA1, bA2 = _nms_pair(swA, *st[0], fic, big)
        swB, a1B, t2B, bB1, bB2 = _nms_pair(swB, *st[1], fic, big)
        omA1 = (oiota == cntA) & a1A
        omA2 = (oiota == cntA + 1) & t2A
        omB1 = (oiota == cntB) & a1B
        omB2 = (oiota == cntB + 1) & t2B
        oA0 = jnp.where(omA2, bA2[0], jnp.where(omA1, bA1[0], oA0))
        oA1 = jnp.where(omA2, bA2[1], jnp.where(omA1, bA1[1], oA1))
        oA2 = jnp.where(omA2, bA2[2], jnp.where(omA1, bA1[2], oA2))
        oA3 = jnp.where(omA2, bA2[3], jnp.where(omA1, bA1[3], oA3))
        oB0 = jnp.where(omB2, bB2[0], jnp.where(omB1, bB1[0], oB0))
        oB1 = jnp.where(omB2, bB2[1], jnp.where(omB1, bB1[1], oB1))
        oB2 = jnp.where(omB2, bB2[2], jnp.where(omB1, bB1[2], oB2))
        oB3 = jnp.where(omB2, bB2[3], jnp.where(omB1, bB1[3], oB3))
        cntA = cntA + a1A.astype(jnp.int32) + t2A.astype(jnp.int32)
        cntB = cntB + a1B.astype(jnp.int32) + t2B.astype(jnp.int32)
        alA = a1A.astype(jnp.int32)
        alB = a1B.astype(jnp.int32)
        return (swA, swB, cntA, cntB, alA, alB,
                oA0, oA1, oA2, oA3, oB0, oB1, oB2, oB3)

    init = (swA0, swB0, one11, one11, one11 + 1, one11 + 1) + (zeros8,) * 8
    res = lax.while_loop(cond, body, init)
    for b in range(2):
        for c in range(4):
            out_ref[b, c] = res[6 + 4 * b + c]


def kernel(rpn_class, rpn_bbox, anchors):
    B = rpn_class.shape[0]
    pad = N_PAD - N_IN
    scores = jnp.pad(rpn_class[:, :, 1], ((0, 0), (0, pad)),
                     constant_values=-1.0).reshape(B, ROWS, 128)
    deltas = jnp.pad(jnp.transpose(rpn_bbox, (0, 2, 1)),
                     ((0, 0), (0, 0), (0, pad))).reshape(B, 4, ROWS, 128)
    anc = jnp.pad(anchors.T, ((0, 0), (0, pad))).reshape(4, ROWS, 128)

    out = pl.pallas_call(
        _proposal_kernel,
        in_specs=[
            pl.BlockSpec((B, ROWS, 128), lambda: (0, 0, 0)),
            pl.BlockSpec((B, 4, ROWS, 128), lambda: (0, 0, 0, 0)),
            pl.BlockSpec((4, ROWS, 128), lambda: (0, 0, 0)),
        ],
        out_specs=pl.BlockSpec((B, 4, 8, 128), lambda: (0, 0, 0, 0)),
        out_shape=jax.ShapeDtypeStruct((B, 4, 8, 128), jnp.float32),
        scratch_shapes=[
            pltpu.VMEM((5 * ROWS, 128), jnp.float32),
            pltpu.VMEM((5 * ROWS, 128), jnp.float32),
            pltpu.VMEM((ROWS, 128), jnp.int32),
            pltpu.VMEM((ROWS, 128), jnp.int32),
            pltpu.VMEM((5 * CPLANE, 128), jnp.float32),
            pltpu.VMEM((5 * CPLANE, 128), jnp.float32),
        ],
    )(scores, deltas, anc)
    return out.reshape(B, 4, 1024)[:, :, :NUM_OUT].transpose(0, 2, 1)
